# Initial kernel scaffold; baseline (speedup 1.0000x reference)
import jax
import jax.numpy as jnp
from jax.experimental import pallas as pl


def kernel(input_H, input_I, adj, weight_H, weight_I, bias1, bias2):
    raise NotImplementedError("write your pallas kernel here")


# speedup vs baseline: 1.7284x; 1.7284x over previous
import jax
import jax.numpy as jnp
from jax.experimental import pallas as pl
from jax.experimental.pallas import tpu as pltpu

N = 10000
D = 128
D2 = 2 * D
BM = 400
NPAD = 10240
SPLIT = 9984
TAIL = NPAD - SPLIT
TAIL_VALID = N - SPLIT


def _support_kernel(x_ref, w_ref, s_ref):
    s_ref[...] = jax.lax.dot(
        x_ref[...], w_ref[...],
        precision=jax.lax.Precision.HIGHEST,
        preferred_element_type=jnp.float32,
    )


def _agg_kernel(adj_ref, s_ref, b_ref, o_ref):
    main = jax.lax.dot(adj_ref[:, :SPLIT], s_ref[:SPLIT, :],
                       preferred_element_type=jnp.float32)
    mask = jax.lax.broadcasted_iota(jnp.int32, (BM, TAIL), 1) < TAIL_VALID
    tail = jax.lax.dot(jnp.where(mask, adj_ref[:, SPLIT:], 0.0),
                       s_ref[SPLIT:, :],
                       preferred_element_type=jnp.float32)
    o_ref[...] = b_ref[...] + main + tail


def kernel(input_H, input_I, adj, weight_H, weight_I, bias1, bias2):
    w2 = jnp.block([[weight_H, weight_I], [-weight_I, weight_H]])
    x = jnp.concatenate([input_H, input_I], axis=1)
    x = jnp.pad(x, ((0, NPAD - N), (0, 0)))

    s = pl.pallas_call(
        _support_kernel,
        grid=(NPAD // 2048,),
        in_specs=[
            pl.BlockSpec((2048, D2), lambda i: (i, 0)),
            pl.BlockSpec((D2, D2), lambda i: (0, 0)),
        ],
        out_specs=pl.BlockSpec((2048, D2), lambda i: (i, 0)),
        out_shape=jax.ShapeDtypeStruct((NPAD, D2), jnp.float32),
    )(x, w2)

    b = jnp.concatenate([bias1, bias2]).reshape(1, D2)

    out = pl.pallas_call(
        _agg_kernel,
        grid=(N // BM,),
        in_specs=[
            pl.BlockSpec((BM, NPAD), lambda i: (i, 0)),
            pl.BlockSpec((NPAD, D2), lambda i: (0, 0)),
            pl.BlockSpec((1, D2), lambda i: (0, 0)),
        ],
        out_specs=pl.BlockSpec((BM, D2), lambda i: (i, 0)),
        out_shape=jax.ShapeDtypeStruct((N, D2), jnp.float32),
        compiler_params=pltpu.CompilerParams(
            dimension_semantics=("parallel",),
        ),
    )(adj, s, b)

    return out[:, :D], out[:, D:]


# speedup vs baseline: 1.8200x; 1.0530x over previous
import jax
import jax.numpy as jnp
from jax.experimental import pallas as pl
from jax.experimental.pallas import tpu as pltpu

N = 10000
D = 128
D2 = 2 * D
BM = 400
BS = 2048
NPAD = 10240
SPLIT = 9984
TAIL = NPAD - SPLIT
TAIL_VALID = N - SPLIT


def _support_kernel(h_ref, i_ref, wt_ref, wb_ref, s_ref):
    i = pl.program_id(0)
    s = jax.lax.dot(h_ref[...], wt_ref[...],
                    precision=jax.lax.Precision.HIGHEST,
                    preferred_element_type=jnp.float32)
    s += jax.lax.dot(i_ref[...], wb_ref[...],
                     precision=jax.lax.Precision.HIGHEST,
                     preferred_element_type=jnp.float32)
    row = i * BS + jax.lax.broadcasted_iota(jnp.int32, (BS, D2), 0)
    s_ref[...] = jnp.where(row < N, s, 0.0)


def _agg_kernel(adj_ref, s_ref, b_ref, oh_ref, oi_ref):
    main = jax.lax.dot(adj_ref[:, :SPLIT], s_ref[:SPLIT, :],
                       preferred_element_type=jnp.float32)
    mask = jax.lax.broadcasted_iota(jnp.int32, (BM, TAIL), 1) < TAIL_VALID
    tail = jax.lax.dot(jnp.where(mask, adj_ref[:, SPLIT:], 0.0),
                       s_ref[SPLIT:, :],
                       preferred_element_type=jnp.float32)
    res = b_ref[...] + main + tail
    oh_ref[...] = res[:, :D]
    oi_ref[...] = res[:, D:]


def kernel(input_H, input_I, adj, weight_H, weight_I, bias1, bias2):
    w_top = jnp.concatenate([weight_H, weight_I], axis=1)
    w_bot = jnp.concatenate([-weight_I, weight_H], axis=1)

    s = pl.pallas_call(
        _support_kernel,
        grid=(NPAD // BS,),
        in_specs=[
            pl.BlockSpec((BS, D), lambda i: (i, 0)),
            pl.BlockSpec((BS, D), lambda i: (i, 0)),
            pl.BlockSpec((D, D2), lambda i: (0, 0)),
            pl.BlockSpec((D, D2), lambda i: (0, 0)),
        ],
        out_specs=pl.BlockSpec((BS, D2), lambda i: (i, 0)),
        out_shape=jax.ShapeDtypeStruct((NPAD, D2), jnp.float32),
    )(input_H, input_I, w_top, w_bot)

    b = jnp.concatenate([bias1, bias2]).reshape(1, D2)

    out_h, out_i = pl.pallas_call(
        _agg_kernel,
        grid=(N // BM,),
        in_specs=[
            pl.BlockSpec((BM, NPAD), lambda i: (i, 0)),
            pl.BlockSpec((NPAD, D2), lambda i: (0, 0)),
            pl.BlockSpec((1, D2), lambda i: (0, 0)),
        ],
        out_specs=[
            pl.BlockSpec((BM, D), lambda i: (i, 0)),
            pl.BlockSpec((BM, D), lambda i: (i, 0)),
        ],
        out_shape=[
            jax.ShapeDtypeStruct((N, D), jnp.float32),
            jax.ShapeDtypeStruct((N, D), jnp.float32),
        ],
        compiler_params=pltpu.CompilerParams(
            dimension_semantics=("arbitrary",),
        ),
    )(adj, s, b)

    return out_h, out_i


# speedup vs baseline: 1.8335x; 1.0074x over previous
import jax
import jax.numpy as jnp
from jax.experimental import pallas as pl
from jax.experimental.pallas import tpu as pltpu

N = 10000
D = 128
D2 = 2 * D
BM = 320
NPAD = 10240
SPLIT = 9984
TAIL = NPAD - SPLIT
TAIL_VALID = N - SPLIT


def _fused_kernel(adj_ref, h_ref, ii_ref, wt_ref, wb_ref, b_ref,
                  oh_ref, oi_ref, s_ref):
    i = pl.program_id(0)

    @pl.when(i == 0)
    def _():
        def body(c, _):
            r0 = c * 2000
            s = jax.lax.dot(h_ref[pl.ds(r0, 2000), :], wt_ref[...],
                            precision=jax.lax.Precision.HIGHEST,
                            preferred_element_type=jnp.float32)
            s += jax.lax.dot(ii_ref[pl.ds(r0, 2000), :], wb_ref[...],
                             precision=jax.lax.Precision.HIGHEST,
                             preferred_element_type=jnp.float32)
            s_ref[pl.ds(r0, 2000), :] = s
            return 0

        jax.lax.fori_loop(0, N // 2000, body, 0)
        s_ref[N:, :] = jnp.zeros((NPAD - N, D2), jnp.float32)

    @pl.when(i > 0)
    def _():
        main = jax.lax.dot(adj_ref[:, :SPLIT], s_ref[:SPLIT, :],
                           preferred_element_type=jnp.float32)
        mask = jax.lax.broadcasted_iota(jnp.int32, (BM, TAIL), 1) < TAIL_VALID
        tail = jax.lax.dot(jnp.where(mask, adj_ref[:, SPLIT:], 0.0),
                           s_ref[SPLIT:, :],
                           preferred_element_type=jnp.float32)
        res = b_ref[...] + main + tail
        oh_ref[...] = res[:, :D]
        oi_ref[...] = res[:, D:]


def kernel(input_H, input_I, adj, weight_H, weight_I, bias1, bias2):
    w_top = jnp.concatenate([weight_H, weight_I], axis=1)
    w_bot = jnp.concatenate([-weight_I, weight_H], axis=1)
    b = jnp.concatenate([bias1, bias2]).reshape(1, D2)

    def _blk(i):
        j = jnp.maximum(i - 1, 0)
        return (j, 0)

    out_h, out_i = pl.pallas_call(
        _fused_kernel,
        grid=((N + BM - 1) // BM + 1,),
        in_specs=[
            pl.BlockSpec((BM, NPAD), _blk),
            pl.BlockSpec((N, D), lambda i: (0, 0)),
            pl.BlockSpec((N, D), lambda i: (0, 0)),
            pl.BlockSpec((D, D2), lambda i: (0, 0)),
            pl.BlockSpec((D, D2), lambda i: (0, 0)),
            pl.BlockSpec((1, D2), lambda i: (0, 0)),
        ],
        out_specs=[
            pl.BlockSpec((BM, D), _blk),
            pl.BlockSpec((BM, D), _blk),
        ],
        out_shape=[
            jax.ShapeDtypeStruct((N, D), jnp.float32),
            jax.ShapeDtypeStruct((N, D), jnp.float32),
        ],
        scratch_shapes=[pltpu.VMEM((NPAD, D2), jnp.float32)],
        compiler_params=pltpu.CompilerParams(
            dimension_semantics=("arbitrary",),
        ),
    )(adj, input_H, input_I, w_top, w_bot, b)

    return out_h, out_i


# speedup vs baseline: 1.8357x; 1.0012x over previous
import jax
import jax.numpy as jnp
from jax.experimental import pallas as pl
from jax.experimental.pallas import tpu as pltpu

N = 10000
D = 128
D2 = 2 * D
BM = 256
NPAD = 10240
SPLIT = 9984
TAIL = NPAD - SPLIT
TAIL_VALID = N - SPLIT


def _fused_kernel(adj_ref, h_ref, ii_ref, wt_ref, wb_ref, b_ref,
                  oh_ref, oi_ref, s_ref):
    i = pl.program_id(0)

    @pl.when(i == 0)
    def _():
        def body(c, _):
            r0 = c * 2000
            s = jax.lax.dot(h_ref[pl.ds(r0, 2000), :], wt_ref[...],
                            precision=jax.lax.Precision.HIGHEST,
                            preferred_element_type=jnp.float32)
            s += jax.lax.dot(ii_ref[pl.ds(r0, 2000), :], wb_ref[...],
                             precision=jax.lax.Precision.HIGHEST,
                             preferred_element_type=jnp.float32)
            s_ref[pl.ds(r0, 2000), :] = s
            return 0

        jax.lax.fori_loop(0, N // 2000, body, 0)
        s_ref[N:, :] = jnp.zeros((NPAD - N, D2), jnp.float32)

    @pl.when(i > 0)
    def _():
        main = jax.lax.dot(adj_ref[:, :SPLIT], s_ref[:SPLIT, :],
                           preferred_element_type=jnp.float32)
        mask = jax.lax.broadcasted_iota(jnp.int32, (BM, TAIL), 1) < TAIL_VALID
        tail = jax.lax.dot(jnp.where(mask, adj_ref[:, SPLIT:], 0.0),
                           s_ref[SPLIT:, :],
                           preferred_element_type=jnp.float32)
        res = b_ref[...] + main + tail
        oh_ref[...] = res[:, :D]
        oi_ref[...] = res[:, D:]


def kernel(input_H, input_I, adj, weight_H, weight_I, bias1, bias2):
    w_top = jnp.concatenate([weight_H, weight_I], axis=1)
    w_bot = jnp.concatenate([-weight_I, weight_H], axis=1)
    b = jnp.concatenate([bias1, bias2]).reshape(1, D2)

    def _blk(i):
        j = jnp.maximum(i - 1, 0)
        return (j, 0)

    out_h, out_i = pl.pallas_call(
        _fused_kernel,
        grid=((N + BM - 1) // BM + 1,),
        in_specs=[
            pl.BlockSpec((BM, NPAD), _blk),
            pl.BlockSpec((N, D), lambda i: (0, 0)),
            pl.BlockSpec((N, D), lambda i: (0, 0)),
            pl.BlockSpec((D, D2), lambda i: (0, 0)),
            pl.BlockSpec((D, D2), lambda i: (0, 0)),
            pl.BlockSpec((1, D2), lambda i: (0, 0)),
        ],
        out_specs=[
            pl.BlockSpec((BM, D), _blk),
            pl.BlockSpec((BM, D), _blk),
        ],
        out_shape=[
            jax.ShapeDtypeStruct((N, D), jnp.float32),
            jax.ShapeDtypeStruct((N, D), jnp.float32),
        ],
        scratch_shapes=[pltpu.VMEM((NPAD, D2), jnp.float32)],
        compiler_params=pltpu.CompilerParams(
            dimension_semantics=("arbitrary",),
        ),
    )(adj, input_H, input_I, w_top, w_bot, b)

    return out_h, out_i


# speedup vs baseline: 2.0211x; 1.1009x over previous
import jax
import jax.numpy as jnp
from jax.experimental import pallas as pl
from jax.experimental.pallas import tpu as pltpu

N = 10000
D = 128
D2 = 2 * D
BM = 256
NPAD = 10240
SPLIT = 9984
TAIL = NPAD - SPLIT
TAIL_VALID = N - SPLIT


def _fused_kernel(adj_ref, h_ref, ii_ref, wt_ref, wb_ref, b_ref,
                  oh_ref, oi_ref, s_ref):
    i = pl.program_id(0)

    @pl.when(i == 0)
    def _():
        def body(c, _):
            r0 = c * 2000
            s = jax.lax.dot(h_ref[pl.ds(r0, 2000), :], wt_ref[...],
                            preferred_element_type=jnp.float32)
            s += jax.lax.dot(ii_ref[pl.ds(r0, 2000), :], wb_ref[...],
                             preferred_element_type=jnp.float32)
            s_ref[pl.ds(r0, 2000), :] = s
            return 0

        jax.lax.fori_loop(0, N // 2000, body, 0)
        s_ref[N:, :] = jnp.zeros((NPAD - N, D2), jnp.float32)

    @pl.when(i > 0)
    def _():
        main = jax.lax.dot(adj_ref[:, :SPLIT], s_ref[:SPLIT, :],
                           preferred_element_type=jnp.float32)
        mask = jax.lax.broadcasted_iota(jnp.int32, (BM, TAIL), 1) < TAIL_VALID
        tail = jax.lax.dot(jnp.where(mask, adj_ref[:, SPLIT:], 0.0),
                           s_ref[SPLIT:, :],
                           preferred_element_type=jnp.float32)
        res = b_ref[...] + main + tail
        oh_ref[...] = res[:, :D]
        oi_ref[...] = res[:, D:]


def kernel(input_H, input_I, adj, weight_H, weight_I, bias1, bias2):
    w_top = jnp.concatenate([weight_H, weight_I], axis=1)
    w_bot = jnp.concatenate([-weight_I, weight_H], axis=1)
    b = jnp.concatenate([bias1, bias2]).reshape(1, D2)

    def _blk(i):
        j = jnp.maximum(i - 1, 0)
        return (j, 0)

    out_h, out_i = pl.pallas_call(
        _fused_kernel,
        grid=((N + BM - 1) // BM + 1,),
        in_specs=[
            pl.BlockSpec((BM, NPAD), _blk),
            pl.BlockSpec((N, D), lambda i: (0, 0)),
            pl.BlockSpec((N, D), lambda i: (0, 0)),
            pl.BlockSpec((D, D2), lambda i: (0, 0)),
            pl.BlockSpec((D, D2), lambda i: (0, 0)),
            pl.BlockSpec((1, D2), lambda i: (0, 0)),
        ],
        out_specs=[
            pl.BlockSpec((BM, D), _blk),
            pl.BlockSpec((BM, D), _blk),
        ],
        out_shape=[
            jax.ShapeDtypeStruct((N, D), jnp.float32),
            jax.ShapeDtypeStruct((N, D), jnp.float32),
        ],
        scratch_shapes=[pltpu.VMEM((NPAD, D2), jnp.float32)],
        compiler_params=pltpu.CompilerParams(
            dimension_semantics=("arbitrary",),
        ),
    )(adj, input_H, input_I, w_top, w_bot, b)

    return out_h, out_i
